# baseline (speedup 1.0000x reference)
import functools

import jax
import jax.numpy as jnp
from jax import lax
from jax.experimental import pallas as pl
from jax.experimental.pallas import tpu as pltpu
from jax.experimental.pallas import tpu_sc as plsc

PASSES = 4
NUM_EDGE_SETS = 2
B, N, D, E = 4, 10000, 150, 320000
NP = 10240
DP = 160
EG = NUM_EDGE_SETS * E
NC, NS, L = 2, 16, 16
EPT = EG // NS
CH = 80
NCHUNK = EPT // CH
SB = 25
NSC = NCHUNK // SB
RPT = NP // NS
SPMOD = 3
GPC = B // NC


def _mp_kernel(nodes_hbm, srcg_hbm, srcl_hbm, dstl_hbm, nscr_hbm,
               inc_shared, state_rep, src_blk, srcl_blk, dst_blk,
               rows0, rows1, rows2,
               gsem0, gsem1, gsem2, ssem0, ssem1, ssem2):
  c = lax.axis_index("c")
  t = lax.axis_index("s")
  rows = (rows0, rows1, rows2)
  gsem = (gsem0, gsem1, gsem2)
  ssem = (ssem0, ssem1, ssem2)

  for gi in range(GPC):
    g = c * GPC + gi
    for p in range(PASSES):
      nsrc = nodes_hbm if p == 0 else nscr_hbm

      lr00 = t * RPT
      pltpu.sync_copy(nsrc.at[pl.ds(g * NP + lr00, RPT)],
                      inc_shared.at[pl.ds(lr00, RPT)])
      pltpu.sync_copy(nsrc.at[pl.ds(g * NP + lr00, RPT)],
                      state_rep.at[pl.ds(lr00, RPT)])
      plsc.subcore_barrier()

      erow = g * (EG // CH) + t * (EPT // CH)

      @pl.loop(0, NSC)
      def _schunk(s):
        r0 = erow + s * SB
        pltpu.sync_copy(srcg_hbm.at[pl.ds(r0, SB)], src_blk)
        pltpu.sync_copy(srcl_hbm.at[pl.ds(r0, SB)], srcl_blk)
        pltpu.sync_copy(dstl_hbm.at[pl.ds(r0, SB)], dst_blk)

        def _gather(j, b):
          if j % SPMOD == SPMOD - 1:
            return pltpu.async_copy(state_rep.at[srcl_blk.at[j]],
                                    rows[b], gsem[b])
          return pltpu.async_copy(nsrc.at[src_blk.at[j]], rows[b], gsem[b])

        gd = [None] * SB
        sd = [None] * SB
        gd[0] = _gather(0, 0)
        gd[1] = _gather(1, 1)
        for j in range(SB):
          b = j % 3
          gd[j].wait()
          sd[j] = pltpu.async_copy(rows[b], inc_shared.at[dst_blk.at[j]],
                                   ssem[b], add=True)
          if j + 2 < SB:
            if j >= 1:
              sd[j - 1].wait()
            gd[j + 2] = _gather(j + 2, (j + 2) % 3)
        sd[SB - 3].wait()
        sd[SB - 2].wait()
        sd[SB - 1].wait()

      plsc.subcore_barrier()

      pltpu.sync_copy(inc_shared.at[pl.ds(lr00, RPT)],
                      nscr_hbm.at[pl.ds(g * NP + lr00, RPT)])


def _readout_kernel(nfin_ref, ptype_ref, w1a_ref, w1b_ref, b1_ref,
                    w2_ref, b2_ref, w3_ref, b3_ref, out_ref):
  g = jnp.sum(nfin_ref[...].astype(jnp.float32), axis=1)[:, :D]
  g = jnp.log(g)
  g = jnp.where(jnp.isnan(g), 0.0, g)
  g = jnp.maximum(g, 0.0)
  x = (jnp.dot(g, w1a_ref[...].T, preferred_element_type=jnp.float32)
       + ptype_ref[...] * w1b_ref[...].T + b1_ref[...])
  x = jnp.where(x > 0, x, 0.01 * x)
  x = jnp.dot(x, w2_ref[...].T, preferred_element_type=jnp.float32) + b2_ref[...]
  x = jnp.where(x > 0, x, 0.01 * x)
  x = jnp.dot(x, w3_ref[...].T, preferred_element_type=jnp.float32) + b3_ref[...]
  out_ref[...] = x


def kernel(nodesBatch, backwards_edgeBatch, problemTypeBatch,
           W1, b1, W2, b2, W3, b3):
  nodes_pad = jnp.pad(nodesBatch, ((0, 0), (0, NP - N), (0, DP - D)))
  nodes_pad = nodes_pad.reshape(B * NP, DP).astype(jnp.bfloat16)
  dst_l = backwards_edgeBatch[..., 0].reshape(B * EG // CH, CH)
  src_l = backwards_edgeBatch[..., 1].reshape(B * EG // CH, CH)
  src_g = (backwards_edgeBatch[..., 1]
           + (jnp.arange(B, dtype=jnp.int32) * NP)[:, None, None]
           ).reshape(B * EG // CH, CH)

  mesh = plsc.VectorSubcoreMesh(core_axis_name="c", subcore_axis_name="s",
                                num_cores=NC, num_subcores=NS)
  mp = pl.kernel(
      _mp_kernel,
      out_type=jax.ShapeDtypeStruct((B * NP, DP), jnp.bfloat16),
      mesh=mesh,
      compiler_params=pltpu.CompilerParams(use_tc_tiling_on_sc=False),
      scratch_types=[
          pltpu.VMEM_SHARED((NP, DP), jnp.bfloat16),
          pltpu.VMEM_SHARED((NP, DP), jnp.bfloat16),
          pltpu.VMEM((SB, CH), jnp.int32),
          pltpu.VMEM((SB, CH), jnp.int32),
          pltpu.VMEM((SB, CH), jnp.int32),
          pltpu.VMEM((CH, DP), jnp.bfloat16),
          pltpu.VMEM((CH, DP), jnp.bfloat16),
          pltpu.VMEM((CH, DP), jnp.bfloat16),
          pltpu.SemaphoreType.DMA,
          pltpu.SemaphoreType.DMA,
          pltpu.SemaphoreType.DMA,
          pltpu.SemaphoreType.DMA,
          pltpu.SemaphoreType.DMA,
          pltpu.SemaphoreType.DMA,
      ],
  )
  nfin = mp(nodes_pad, src_g, src_l, dst_l).reshape(B, NP, DP)

  out = pl.pallas_call(
      _readout_kernel,
      out_shape=jax.ShapeDtypeStruct((B, 10), jnp.float32),
  )(nfin, problemTypeBatch, W1[:, :D], W1[:, D:], b1, W2, b2, W3, b3)
  return out


# speedup vs baseline: 1.2023x; 1.2023x over previous
import functools

import jax
import jax.numpy as jnp
from jax import lax
from jax.experimental import pallas as pl
from jax.experimental.pallas import tpu as pltpu
from jax.experimental.pallas import tpu_sc as plsc

PASSES = 4
NUM_EDGE_SETS = 2
B, N, D, E = 4, 10000, 150, 320000
NP = 10240
DP = 160
EG = NUM_EDGE_SETS * E
NC, NS, L = 2, 16, 16
EPT = EG // NS
CH = 80
NCHUNK = EPT // CH
SB = 25
NSC = NCHUNK // SB
RPT = NP // NS
GPC = B // NC


def _mp_kernel(nodes_hbm, idx_hbm, nscr_hbm,
               inc_shared, iblk0, iblk1, rows0, rows1, rows2,
               gsem0, gsem1, gsem2, ssem0, ssem1, ssem2, isem0, isem1):
  c = lax.axis_index("c")
  t = lax.axis_index("s")
  rows = (rows0, rows1, rows2)
  gsem = (gsem0, gsem1, gsem2)
  ssem = (ssem0, ssem1, ssem2)

  for gi in range(GPC):
    g = c * GPC + gi
    for p in range(PASSES):
      nsrc = nodes_hbm if p == 0 else nscr_hbm

      lr00 = t * RPT
      pltpu.sync_copy(nsrc.at[pl.ds(g * NP + lr00, RPT)],
                      inc_shared.at[pl.ds(lr00, RPT)])
      plsc.subcore_barrier()

      erow = g * (EG // CH) + t * (EPT // CH)
      iblk = (iblk0, iblk1)
      isem = (isem0, isem1)

      def _process(blk):
        gd = [None] * SB
        sd = [None] * SB
        gd[0] = pltpu.async_copy(nsrc.at[blk.at[0, 0]], rows[0], gsem[0])
        gd[1] = pltpu.async_copy(nsrc.at[blk.at[1, 0]], rows[1], gsem[1])
        for j in range(SB):
          b = j % 3
          gd[j].wait()
          sd[j] = pltpu.async_copy(rows[b], inc_shared.at[blk.at[j, 1]],
                                   ssem[b], add=True)
          if j + 2 < SB:
            if j >= 1:
              sd[j - 1].wait()
            gd[j + 2] = pltpu.async_copy(nsrc.at[blk.at[j + 2, 0]],
                                         rows[(j + 2) % 3], gsem[(j + 2) % 3])
        sd[SB - 3].wait()
        sd[SB - 2].wait()
        sd[SB - 1].wait()

      pltpu.async_copy(idx_hbm.at[pl.ds(erow, SB)], iblk0, isem0)

      @pl.loop(0, NSC, step=2)
      def _schunk(s):
        pltpu.make_async_copy(idx_hbm.at[pl.ds(erow + s * SB, SB)],
                              iblk0, isem0).wait()
        pltpu.async_copy(idx_hbm.at[pl.ds(erow + (s + 1) * SB, SB)],
                         iblk1, isem1)
        _process(iblk0)
        pltpu.make_async_copy(idx_hbm.at[pl.ds(erow + (s + 1) * SB, SB)],
                              iblk1, isem1).wait()

        @pl.when(s + 2 < NSC)
        def _prefetch():
          pltpu.async_copy(idx_hbm.at[pl.ds(erow + (s + 2) * SB, SB)],
                           iblk0, isem0)

        _process(iblk1)

      plsc.subcore_barrier()

      pltpu.sync_copy(inc_shared.at[pl.ds(lr00, RPT)],
                      nscr_hbm.at[pl.ds(g * NP + lr00, RPT)])


def _readout_kernel(nfin_ref, ptype_ref, w1a_ref, w1b_ref, b1_ref,
                    w2_ref, b2_ref, w3_ref, b3_ref, out_ref):
  g = jnp.sum(nfin_ref[...].astype(jnp.float32), axis=1)[:, :D]
  g = jnp.log(g)
  g = jnp.where(jnp.isnan(g), 0.0, g)
  g = jnp.maximum(g, 0.0)
  x = (jnp.dot(g, w1a_ref[...].T, preferred_element_type=jnp.float32)
       + ptype_ref[...] * w1b_ref[...].T + b1_ref[...])
  x = jnp.where(x > 0, x, 0.01 * x)
  x = jnp.dot(x, w2_ref[...].T, preferred_element_type=jnp.float32) + b2_ref[...]
  x = jnp.where(x > 0, x, 0.01 * x)
  x = jnp.dot(x, w3_ref[...].T, preferred_element_type=jnp.float32) + b3_ref[...]
  out_ref[...] = x


def kernel(nodesBatch, backwards_edgeBatch, problemTypeBatch,
           W1, b1, W2, b2, W3, b3):
  nodes_pad = jnp.pad(nodesBatch, ((0, 0), (0, NP - N), (0, DP - D)))
  nodes_pad = nodes_pad.reshape(B * NP, DP).astype(jnp.bfloat16)
  dst_l = backwards_edgeBatch[..., 0].reshape(B * EG // CH, CH)
  src_g = (backwards_edgeBatch[..., 1]
           + (jnp.arange(B, dtype=jnp.int32) * NP)[:, None, None]
           ).reshape(B * EG // CH, CH)
  idx_comb = jnp.stack([src_g, dst_l], axis=1)

  mesh = plsc.VectorSubcoreMesh(core_axis_name="c", subcore_axis_name="s",
                                num_cores=NC, num_subcores=NS)
  mp = pl.kernel(
      _mp_kernel,
      out_type=jax.ShapeDtypeStruct((B * NP, DP), jnp.bfloat16),
      mesh=mesh,
      compiler_params=pltpu.CompilerParams(use_tc_tiling_on_sc=False),
      scratch_types=[
          pltpu.VMEM_SHARED((NP, DP), jnp.bfloat16),
          pltpu.VMEM((SB, 2, CH), jnp.int32),
          pltpu.VMEM((SB, 2, CH), jnp.int32),
          pltpu.VMEM((CH, DP), jnp.bfloat16),
          pltpu.VMEM((CH, DP), jnp.bfloat16),
          pltpu.VMEM((CH, DP), jnp.bfloat16),
          pltpu.SemaphoreType.DMA,
          pltpu.SemaphoreType.DMA,
          pltpu.SemaphoreType.DMA,
          pltpu.SemaphoreType.DMA,
          pltpu.SemaphoreType.DMA,
          pltpu.SemaphoreType.DMA,
          pltpu.SemaphoreType.DMA,
          pltpu.SemaphoreType.DMA,
      ],
  )
  nfin = mp(nodes_pad, idx_comb).reshape(B, NP, DP)

  out = pl.pallas_call(
      _readout_kernel,
      out_shape=jax.ShapeDtypeStruct((B, 10), jnp.float32),
  )(nfin, problemTypeBatch, W1[:, :D], W1[:, D:], b1, W2, b2, W3, b3)
  return out


# speedup vs baseline: 1.2155x; 1.0110x over previous
import functools

import jax
import jax.numpy as jnp
from jax import lax
from jax.experimental import pallas as pl
from jax.experimental.pallas import tpu as pltpu
from jax.experimental.pallas import tpu_sc as plsc

PASSES = 4
NUM_EDGE_SETS = 2
B, N, D, E = 4, 10000, 150, 320000
NP = 10240
DP = 160
EG = NUM_EDGE_SETS * E
NC, NS, L = 2, 16, 16
EPT = EG // NS
CH = 80
NCHUNK = EPT // CH
SB = 25
NSC = NCHUNK // SB
RPT = NP // NS
GPC = B // NC


def _mp_kernel(nodes_hbm, idx_hbm, nscr_hbm,
               inc_shared, iblk0, iblk1, rows0, rows1, rows2,
               gsem0, gsem1, gsem2, ssem0, ssem1, ssem2, isem0, isem1):
  c = lax.axis_index("c")
  t = lax.axis_index("s")
  rows = (rows0, rows1, rows2)
  gsem = (gsem0, gsem1, gsem2)
  ssem = (ssem0, ssem1, ssem2)

  for gi in range(GPC):
    g = c * GPC + gi
    for p in range(PASSES):
      nsrc = nodes_hbm if p == 0 else nscr_hbm

      lr00 = t * RPT
      pltpu.sync_copy(nsrc.at[pl.ds(g * NP + lr00, RPT)],
                      inc_shared.at[pl.ds(lr00, RPT)])
      plsc.subcore_barrier()

      erow = g * (EG // CH) + t * (EPT // CH)
      iblk = (iblk0, iblk1)
      isem = (isem0, isem1)

      def _drain(b):
        pltpu.make_async_copy(rows[b], inc_shared.at[iblk0.at[0, 1]],
                              ssem[b]).wait()

      def _process(blk, entry_drains):
        gd = [None] * SB
        sd = [None] * SB
        entry_drains(1)
        entry_drains(2)
        entry_drains(0)
        gd[0] = pltpu.async_copy(nsrc.at[blk.at[0, 0]], rows[0], gsem[0])
        gd[1] = pltpu.async_copy(nsrc.at[blk.at[1, 0]], rows[1], gsem[1])
        for j in range(SB):
          b = j % 3
          gd[j].wait()
          sd[j] = pltpu.async_copy(rows[b], inc_shared.at[blk.at[j, 1]],
                                   ssem[b], add=True)
          if j + 2 < SB:
            if j >= 1:
              sd[j - 1].wait()
            gd[j + 2] = pltpu.async_copy(nsrc.at[blk.at[j + 2, 0]],
                                         rows[(j + 2) % 3], gsem[(j + 2) % 3])
        return sd

      pltpu.async_copy(idx_hbm.at[pl.ds(erow, SB)], iblk0, isem0)

      @pl.loop(0, NSC, step=2)
      def _schunk(s):
        pltpu.make_async_copy(idx_hbm.at[pl.ds(erow + s * SB, SB)],
                              iblk0, isem0).wait()
        pltpu.async_copy(idx_hbm.at[pl.ds(erow + (s + 1) * SB, SB)],
                         iblk1, isem1)

        def _entry0(b):
          @pl.when(s > 0)
          def _():
            _drain(b)

        sd0 = _process(iblk0, _entry0)
        pltpu.make_async_copy(idx_hbm.at[pl.ds(erow + (s + 1) * SB, SB)],
                              iblk1, isem1).wait()

        @pl.when(s + 2 < NSC)
        def _prefetch():
          pltpu.async_copy(idx_hbm.at[pl.ds(erow + (s + 2) * SB, SB)],
                           iblk0, isem0)

        _process(iblk1, lambda b: sd0[SB - 3 + ((b - 1) % 3)].wait())

      _drain(1)
      _drain(2)
      _drain(0)
      plsc.subcore_barrier()

      pltpu.sync_copy(inc_shared.at[pl.ds(lr00, RPT)],
                      nscr_hbm.at[pl.ds(g * NP + lr00, RPT)])


def _readout_kernel(nfin_ref, ptype_ref, w1a_ref, w1b_ref, b1_ref,
                    w2_ref, b2_ref, w3_ref, b3_ref, out_ref):
  g = jnp.sum(nfin_ref[...].astype(jnp.float32), axis=1)[:, :D]
  g = jnp.log(g)
  g = jnp.where(jnp.isnan(g), 0.0, g)
  g = jnp.maximum(g, 0.0)
  x = (jnp.dot(g, w1a_ref[...].T, preferred_element_type=jnp.float32)
       + ptype_ref[...] * w1b_ref[...].T + b1_ref[...])
  x = jnp.where(x > 0, x, 0.01 * x)
  x = jnp.dot(x, w2_ref[...].T, preferred_element_type=jnp.float32) + b2_ref[...]
  x = jnp.where(x > 0, x, 0.01 * x)
  x = jnp.dot(x, w3_ref[...].T, preferred_element_type=jnp.float32) + b3_ref[...]
  out_ref[...] = x


def kernel(nodesBatch, backwards_edgeBatch, problemTypeBatch,
           W1, b1, W2, b2, W3, b3):
  nodes_pad = jnp.pad(nodesBatch, ((0, 0), (0, NP - N), (0, DP - D)))
  nodes_pad = nodes_pad.reshape(B * NP, DP).astype(jnp.bfloat16)
  dst_l = backwards_edgeBatch[..., 0].reshape(B * EG // CH, CH)
  src_g = (backwards_edgeBatch[..., 1]
           + (jnp.arange(B, dtype=jnp.int32) * NP)[:, None, None]
           ).reshape(B * EG // CH, CH)
  idx_comb = jnp.stack([src_g, dst_l], axis=1)

  mesh = plsc.VectorSubcoreMesh(core_axis_name="c", subcore_axis_name="s",
                                num_cores=NC, num_subcores=NS)
  mp = pl.kernel(
      _mp_kernel,
      out_type=jax.ShapeDtypeStruct((B * NP, DP), jnp.bfloat16),
      mesh=mesh,
      compiler_params=pltpu.CompilerParams(use_tc_tiling_on_sc=False),
      scratch_types=[
          pltpu.VMEM_SHARED((NP, DP), jnp.bfloat16),
          pltpu.VMEM((SB, 2, CH), jnp.int32),
          pltpu.VMEM((SB, 2, CH), jnp.int32),
          pltpu.VMEM((CH, DP), jnp.bfloat16),
          pltpu.VMEM((CH, DP), jnp.bfloat16),
          pltpu.VMEM((CH, DP), jnp.bfloat16),
          pltpu.SemaphoreType.DMA,
          pltpu.SemaphoreType.DMA,
          pltpu.SemaphoreType.DMA,
          pltpu.SemaphoreType.DMA,
          pltpu.SemaphoreType.DMA,
          pltpu.SemaphoreType.DMA,
          pltpu.SemaphoreType.DMA,
          pltpu.SemaphoreType.DMA,
      ],
  )
  nfin = mp(nodes_pad, idx_comb).reshape(B, NP, DP)

  out = pl.pallas_call(
      _readout_kernel,
      out_shape=jax.ShapeDtypeStruct((B, 10), jnp.float32),
  )(nfin, problemTypeBatch, W1[:, :D], W1[:, D:], b1, W2, b2, W3, b3)
  return out
